# baseline (speedup 1.0000x reference)
import functools

import jax
import jax.numpy as jnp
import numpy as np
from jax import lax
from jax.experimental import pallas as pl
from jax.experimental.pallas import tpu as pltpu
from jax.experimental.pallas import tpu_sc as plsc

_P = 0.15
_LANES = 128
_NW = 32
_NSPLIT = 4

_tbl_cache = {}


def _tf2x32(k1, k2, x0, x1):
    rots = [np.array([13, 15, 26, 6], dtype=np.uint32),
            np.array([17, 29, 16, 24], dtype=np.uint32)]
    ks = [np.uint32(k1), np.uint32(k2),
          np.uint32(k1) ^ np.uint32(k2) ^ np.uint32(0x1BD11BDA)]
    x0 = (x0 + ks[0]).astype(np.uint32)
    x1 = (x1 + ks[1]).astype(np.uint32)
    kr = [ks[1], ks[2], ks[0]]
    rr = [rots[0], rots[1]]
    for i in range(5):
        for r in rr[0]:
            x0 = (x0 + x1).astype(np.uint32)
            x1 = ((x1 << r) | (x1 >> (np.uint32(32) - r))).astype(np.uint32)
            x1 = x0 ^ x1
        x0 = (x0 + kr[0]).astype(np.uint32)
        x1 = (x1 + kr[1] + np.uint32(i + 1)).astype(np.uint32)
        kr = [kr[1], kr[2], kr[0]]
        rr = [rr[1], rr[0]]
    return x0, x1


def _np_uniform(key, n):
    b1, b2 = _tf2x32(key[0], key[1],
                     np.zeros(n, dtype=np.uint32),
                     np.arange(n, dtype=np.uint32))
    bits = b1 ^ b2
    return ((bits >> np.uint32(9)) | np.uint32(0x3F800000)).view(np.float32) \
        - np.float32(1.0)


def _swap_tables(B, F):
    if (B, F) in _tbl_cache:
        return _tbl_cache[(B, F)]
    n = B * F
    s1, s2 = _tf2x32(np.uint32(0), np.uint32(42),
                     np.zeros(2, dtype=np.uint32),
                     np.arange(2, dtype=np.uint32))
    mask = _np_uniform((s1[0], s2[0]), n) > np.float32(1.0 - _P)
    l1 = np.floor(_np_uniform((s1[1], s2[1]), n) * np.float32(B)).astype(np.int32)
    idx = np.arange(n, dtype=np.int32) + l1 * (mask.astype(np.int32) * F)
    idx = np.where(idx >= n, idx - n, idx)

    epw = n // _NW
    seg = epw // _NSPLIT
    parts = []
    for w in range(_NW):
        lp = np.nonzero(mask[w * epw:(w + 1) * epw])[0].astype(np.int32)
        src = idx[lp + w * epw]
        cuts = np.searchsorted(lp, np.arange(1, _NSPLIT) * seg)
        parts.append((np.split(lp, cuts), np.split(src, cuts)))
    kmax = max(max(len(a) for a in lps) for lps, _ in parts)
    K = max(-(-kmax // _LANES) * _LANES, _LANES)
    src_t = np.empty((_NW, _NSPLIT, K), dtype=np.int32)
    dst_t = np.empty((_NW, _NSPLIT, K), dtype=np.int32)
    for w in range(_NW):
        lps, srcs = parts[w]
        for h in range(_NSPLIT):
            lp, src = lps[h], srcs[h]
            if len(lp) == 0:
                lp = np.array([h * seg], dtype=np.int32)
                src = np.array([w * epw + h * seg], dtype=np.int32)
            pad = K - len(lp)
            dst = lp + (w // 2) * epw
            src_t[w, h] = np.concatenate([src, np.full(pad, src[0], np.int32)])
            dst_t[w, h] = np.concatenate([dst, np.full(pad, dst[0], np.int32)])
    _tbl_cache[(B, F)] = (src_t.reshape(-1), dst_t.reshape(-1), K)
    return _tbl_cache[(B, F)]


@functools.partial(jax.jit, static_argnames=("epw", "K"))
def _swap_call(x_flat, src_tbl, dst_tbl, epw, K):
    n = x_flat.shape[0]
    mesh = plsc.VectorSubcoreMesh(core_axis_name="c", subcore_axis_name="s")
    NS = _NSPLIT
    seg = epw // NS

    @functools.partial(
        pl.kernel,
        out_type=jax.ShapeDtypeStruct((n,), jnp.float32),
        mesh=mesh,
        scratch_types=[
            [pltpu.VMEM((K,), jnp.int32)] * NS,
            [pltpu.VMEM((K,), jnp.int32)] * NS,
            [pltpu.VMEM((K,), jnp.float32)] * NS,
            pltpu.VMEM_SHARED((16 * epw,), jnp.float32),
            [pltpu.SemaphoreType.DMA] * NS,
            pltpu.SemaphoreType.DMA,
            pltpu.SemaphoreType.DMA,
            pltpu.SemaphoreType.DMA,
        ],
    )
    def k(x_hbm, src_hbm, dst_hbm, out_hbm, src_v, dst_v, gath_v, shared,
          sem_g, sem_t, sem_x, sem_o):
        sid = lax.axis_index("s")
        wid = sid * 2 + lax.axis_index("c")
        base = wid * epw
        sbase = sid * epw
        tb = wid * NS * K
        d_src = [pltpu.async_copy(src_hbm.at[pl.ds(tb + h * K, K)], src_v[h],
                                  sem_t) for h in range(NS)]
        d_dst = [pltpu.async_copy(dst_hbm.at[pl.ds(tb + h * K, K)], dst_v[h],
                                  sem_t) for h in range(NS)]
        d_x = pltpu.async_copy(x_hbm.at[pl.ds(base, epw)],
                               shared.at[pl.ds(sbase, epw)], sem_x)
        gathers = []
        for h in range(NS):
            d_src[h].wait()
            gathers.append(
                pltpu.async_copy(x_hbm.at[src_v[h]], gath_v[h], sem_g[h]))
        for d in d_dst:
            d.wait()
        d_x.wait()
        outs = []
        for h in range(NS):
            gathers[h].wait()
            pltpu.sync_copy(gath_v[h], shared.at[dst_v[h]])
            outs.append(
                pltpu.async_copy(shared.at[pl.ds(sbase + h * seg, seg)],
                                 out_hbm.at[pl.ds(base + h * seg, seg)],
                                 sem_o))
        for d in outs:
            d.wait()

    return k(x_flat, src_tbl, dst_tbl)


def kernel(x):
    B, F = x.shape
    n = B * F
    assert n % (_NW * _NSPLIT * _LANES) == 0
    src_t, dst_t, K = _swap_tables(B, F)
    out = _swap_call(x.reshape(-1), jnp.asarray(src_t), jnp.asarray(dst_t),
                     n // _NW, K)
    return out.reshape(B, F)


# speedup vs baseline: 1.0660x; 1.0660x over previous
import functools

import jax
import jax.numpy as jnp
import numpy as np
from jax import lax
from jax.experimental import pallas as pl
from jax.experimental.pallas import tpu as pltpu
from jax.experimental.pallas import tpu_sc as plsc

_P = 0.15
_LANES = 128
_NW = 32
_NSPLIT = 2

_tbl_cache = {}


def _tf2x32(k1, k2, x0, x1):
    rots = [np.array([13, 15, 26, 6], dtype=np.uint32),
            np.array([17, 29, 16, 24], dtype=np.uint32)]
    ks = [np.uint32(k1), np.uint32(k2),
          np.uint32(k1) ^ np.uint32(k2) ^ np.uint32(0x1BD11BDA)]
    x0 = (x0 + ks[0]).astype(np.uint32)
    x1 = (x1 + ks[1]).astype(np.uint32)
    kr = [ks[1], ks[2], ks[0]]
    rr = [rots[0], rots[1]]
    for i in range(5):
        for r in rr[0]:
            x0 = (x0 + x1).astype(np.uint32)
            x1 = ((x1 << r) | (x1 >> (np.uint32(32) - r))).astype(np.uint32)
            x1 = x0 ^ x1
        x0 = (x0 + kr[0]).astype(np.uint32)
        x1 = (x1 + kr[1] + np.uint32(i + 1)).astype(np.uint32)
        kr = [kr[1], kr[2], kr[0]]
        rr = [rr[1], rr[0]]
    return x0, x1


def _np_uniform(key, n):
    b1, b2 = _tf2x32(key[0], key[1],
                     np.zeros(n, dtype=np.uint32),
                     np.arange(n, dtype=np.uint32))
    bits = b1 ^ b2
    return ((bits >> np.uint32(9)) | np.uint32(0x3F800000)).view(np.float32) \
        - np.float32(1.0)


def _swap_tables(B, F):
    if (B, F) in _tbl_cache:
        return _tbl_cache[(B, F)]
    n = B * F
    s1, s2 = _tf2x32(np.uint32(0), np.uint32(42),
                     np.zeros(2, dtype=np.uint32),
                     np.arange(2, dtype=np.uint32))
    mask = _np_uniform((s1[0], s2[0]), n) > np.float32(1.0 - _P)
    l1 = np.floor(_np_uniform((s1[1], s2[1]), n) * np.float32(B)).astype(np.int32)
    idx = np.arange(n, dtype=np.int32) + l1 * (mask.astype(np.int32) * F)
    idx = np.where(idx >= n, idx - n, idx)

    epw = n // _NW
    seg = epw // _NSPLIT
    parts = []
    for w in range(_NW):
        lp = np.nonzero(mask[w * epw:(w + 1) * epw])[0].astype(np.int32)
        src = idx[lp + w * epw]
        cuts = np.searchsorted(lp, np.arange(1, _NSPLIT) * seg)
        parts.append((np.split(lp, cuts), np.split(src, cuts)))
    kmax = max(max(len(a) for a in lps) for lps, _ in parts)
    K = max(-(-kmax // _LANES) * _LANES, _LANES)
    src_t = np.empty((_NW, _NSPLIT, K), dtype=np.int32)
    dst_t = np.empty((_NW, _NSPLIT, K), dtype=np.int32)
    for w in range(_NW):
        lps, srcs = parts[w]
        for h in range(_NSPLIT):
            lp, src = lps[h], srcs[h]
            if len(lp) == 0:
                lp = np.array([h * seg], dtype=np.int32)
                src = np.array([w * epw + h * seg], dtype=np.int32)
            pad = K - len(lp)
            dst = lp + (w // 2) * epw
            src_t[w, h] = np.concatenate([src, np.full(pad, src[0], np.int32)])
            dst_t[w, h] = np.concatenate([dst, np.full(pad, dst[0], np.int32)])
    _tbl_cache[(B, F)] = (src_t.reshape(-1), dst_t.reshape(-1), K)
    return _tbl_cache[(B, F)]


@functools.partial(jax.jit, static_argnames=("epw", "K"))
def _swap_call(x_flat, src_tbl, dst_tbl, epw, K):
    n = x_flat.shape[0]
    mesh = plsc.VectorSubcoreMesh(core_axis_name="c", subcore_axis_name="s")
    NS = _NSPLIT
    seg = epw // NS

    @functools.partial(
        pl.kernel,
        out_type=jax.ShapeDtypeStruct((n,), jnp.float32),
        mesh=mesh,
        scratch_types=[
            [pltpu.VMEM((K,), jnp.int32)] * NS,
            [pltpu.VMEM((K,), jnp.int32)] * NS,
            [pltpu.VMEM((K,), jnp.float32)] * NS,
            pltpu.VMEM_SHARED((16 * epw,), jnp.float32),
            [pltpu.SemaphoreType.DMA] * NS,
            pltpu.SemaphoreType.DMA,
            pltpu.SemaphoreType.DMA,
            pltpu.SemaphoreType.DMA,
        ],
    )
    def k(x_hbm, src_hbm, dst_hbm, out_hbm, src_v, dst_v, gath_v, shared,
          sem_g, sem_t, sem_x, sem_o):
        sid = lax.axis_index("s")
        wid = sid * 2 + lax.axis_index("c")
        base = wid * epw
        sbase = sid * epw
        tb = wid * NS * K
        d_src = [pltpu.async_copy(src_hbm.at[pl.ds(tb + h * K, K)], src_v[h],
                                  sem_t) for h in range(NS)]
        d_dst = [pltpu.async_copy(dst_hbm.at[pl.ds(tb + h * K, K)], dst_v[h],
                                  sem_t) for h in range(NS)]
        d_x = pltpu.async_copy(x_hbm.at[pl.ds(base, epw)],
                               shared.at[pl.ds(sbase, epw)], sem_x)
        gathers = []
        for h in range(NS):
            d_src[h].wait()
            gathers.append(
                pltpu.async_copy(x_hbm.at[src_v[h]], gath_v[h], sem_g[h]))
        for d in d_dst:
            d.wait()
        d_x.wait()
        outs = []
        for h in range(NS):
            gathers[h].wait()
            pltpu.sync_copy(gath_v[h], shared.at[dst_v[h]])
            outs.append(
                pltpu.async_copy(shared.at[pl.ds(sbase + h * seg, seg)],
                                 out_hbm.at[pl.ds(base + h * seg, seg)],
                                 sem_o))
        for d in outs:
            d.wait()

    return k(x_flat, src_tbl, dst_tbl)


def kernel(x):
    B, F = x.shape
    n = B * F
    assert n % (_NW * _NSPLIT * _LANES) == 0
    src_t, dst_t, K = _swap_tables(B, F)
    out = _swap_call(x.reshape(-1), jnp.asarray(src_t), jnp.asarray(dst_t),
                     n // _NW, K)
    return out.reshape(B, F)
